# Initial kernel scaffold; baseline (speedup 1.0000x reference)
import jax
import jax.numpy as jnp
from jax.experimental import pallas as pl


def kernel(x, edge_index, W1, b1, ln_w, ln_b, W2, b2, W3, b3, W4, b4):
    raise NotImplementedError("write your pallas kernel here")


# speedup vs baseline: 5.4757x; 5.4757x over previous
import functools

import jax
import jax.numpy as jnp
from jax import lax
from jax.experimental import pallas as pl
from jax.experimental.pallas import tpu as pltpu
from jax.experimental.pallas import tpu_sc as plsc

N = 10000
NP = 10240
E = 160000
EP = 163840
D_IN = 256
D_H = 512
L = 16
NC = 2
NS = 16
RPT = NP // NS
IB = 128
HIGH = lax.Precision.HIGHEST
F32 = jnp.float32


def _mesh():
    return plsc.VectorSubcoreMesh(core_axis_name="c", subcore_axis_name="s",
                                  num_cores=NC, num_subcores=NS)


def _fill_rows(ref, nrows, val, ncols=L):
    def body(i, carry):
        for j in range(ncols // L):
            ref[i, pl.ds(j * L, L)] = jnp.full((L,), val, F32)
        return carry
    lax.fori_loop(0, nrows, body, 0)


def _sc_degree(dstp):
    ept = EP // (NC * NS)
    nb = ept // IB
    fr = 64

    @functools.partial(
        pl.kernel,
        out_type=(jax.ShapeDtypeStruct((NP, 128), F32),
                  jax.ShapeDtypeStruct((NP, 128), F32)),
        mesh=_mesh(),
        scratch_types=[
            pltpu.VMEM((IB,), jnp.int32),
            pltpu.VMEM((IB, 128), F32),
            pltpu.VMEM((fr, 128), F32),
            pltpu.VMEM_SHARED((NP, 128), F32),
        ],
    )
    def run(dst_hbm, d0_hbm, d1_hbm, idx_v, ones_v, fill_v, acc_sh):
        c = lax.axis_index("c")
        s = lax.axis_index("s")
        _fill_rows(ones_v, IB, 1.0, ncols=128)
        _fill_rows(fill_v, fr, 0.0, ncols=128)
        for r in range(RPT // fr):
            pltpu.sync_copy(fill_v,
                            acc_sh.at[pl.ds(s * RPT + r * fr, fr)])
        plsc.subcore_barrier()
        base = c * (EP // NC) + s * ept

        def batch(k, carry):
            pltpu.sync_copy(dst_hbm.at[pl.ds(base + k * IB, IB)], idx_v)
            pltpu.sync_copy(ones_v, acc_sh.at[idx_v], add=True)
            return carry

        lax.fori_loop(0, nb, batch, 0)
        plsc.subcore_barrier()
        rows = pl.ds(s * RPT, RPT)

        @pl.when(c == 0)
        def _():
            pltpu.sync_copy(acc_sh.at[rows], d0_hbm.at[rows])

        @pl.when(c == 1)
        def _():
            pltpu.sync_copy(acc_sh.at[rows], d1_hbm.at[rows])

    return run(dstp)


def _sc_prop(srcp, dstp2d, tables):
    n = len(tables)
    cpc = n // NC
    ept = EP // NS
    nb = ept // IB
    nbh = nb // 2

    @functools.partial(
        pl.kernel,
        out_type=tuple(jax.ShapeDtypeStruct((NP, 128), F32)
                       for _ in range(n)),
        mesh=_mesh(),
        scratch_types=[
            pltpu.VMEM((IB,), jnp.int32),
            pltpu.VMEM((IB,), jnp.int32),
            pltpu.VMEM((nb, IB), jnp.int32),
            pltpu.VMEM((IB, 128), F32),
            pltpu.VMEM((IB, 128), F32),
            pltpu.VMEM_SHARED((NP, 128), F32),
            pltpu.SemaphoreType.DMA,
            pltpu.SemaphoreType.DMA,
        ],
    )
    def run(src_hbm, dst_hbm, *rest):
        tbls = rest[:n]
        outs = rest[n:2 * n]
        (sidx0, sidx1, didx_v, rows0, rows1,
         acc_sh, sem0, sem1) = rest[2 * n:]
        c = lax.axis_index("c")
        s = lax.axis_index("s")
        rows = pl.ds(s * RPT, RPT)
        base = s * ept
        pltpu.sync_copy(dst_hbm.at[pl.ds(s * nb, nb)], didx_v)

        for cc in range(NC):
            @pl.when(c == cc)
            def _(cc=cc):
                for k in range(cpc):
                    ch = cc * cpc + k
                    tbl = tbls[ch]
                    out = outs[ch]
                    pltpu.sync_copy(tbl.at[rows], acc_sh.at[rows])
                    plsc.subcore_barrier()

                    def batch2(kk, carry, tbl=tbl):
                        e0 = base + (2 * kk) * IB
                        pltpu.sync_copy(src_hbm.at[pl.ds(e0, IB)], sidx0)
                        cpa = pltpu.async_copy(tbl.at[sidx0], rows0, sem0)
                        pltpu.sync_copy(src_hbm.at[pl.ds(e0 + IB, IB)],
                                        sidx1)
                        cpb = pltpu.async_copy(tbl.at[sidx1], rows1, sem1)
                        cpa.wait()
                        pltpu.sync_copy(rows0,
                                        acc_sh.at[didx_v.at[2 * kk]],
                                        add=True)
                        cpb.wait()
                        pltpu.sync_copy(rows1,
                                        acc_sh.at[didx_v.at[2 * kk + 1]],
                                        add=True)
                        return carry

                    lax.fori_loop(0, nbh, batch2, 0)
                    plsc.subcore_barrier()
                    pltpu.sync_copy(acc_sh.at[rows], out.at[rows])
                    plsc.subcore_barrier()

    return run(srcp, dstp2d, *tables)


def _sc_prop_w(srcp, dstp, z16):
    ept = EP // (NC * NS)
    nb = ept // IB

    @functools.partial(
        pl.kernel,
        out_type=(jax.ShapeDtypeStruct((NP, 128), F32),
                  jax.ShapeDtypeStruct((NP, 128), F32)),
        mesh=_mesh(),
        scratch_types=[
            pltpu.VMEM((IB,), jnp.int32),
            pltpu.VMEM((IB,), jnp.int32),
            pltpu.VMEM((IB, 128), F32),
            pltpu.VMEM_SHARED((NP, 128), F32),
            pltpu.SemaphoreType.DMA,
        ],
    )
    def run(src_hbm, dst_hbm, z_hbm, pa_hbm, pb_hbm,
            sidx, didx, zrows_v, acc_sh, sem):
        c = lax.axis_index("c")
        s = lax.axis_index("s")
        rows = pl.ds(s * RPT, RPT)
        pltpu.sync_copy(z_hbm.at[rows], acc_sh.at[rows])
        plsc.subcore_barrier()
        base = c * (EP // NC) + s * ept

        def batch(k, carry):
            e0 = base + k * IB
            pltpu.sync_copy(src_hbm.at[pl.ds(e0, IB)], sidx)
            pltpu.sync_copy(dst_hbm.at[pl.ds(e0, IB)], didx)
            pltpu.async_copy(z_hbm.at[sidx], zrows_v, sem).wait()
            pltpu.sync_copy(zrows_v, acc_sh.at[didx], add=True)
            return carry

        lax.fori_loop(0, nb, batch, 0)
        plsc.subcore_barrier()

        @pl.when(c == 0)
        def _():
            pltpu.sync_copy(acc_sh.at[rows], pa_hbm.at[rows])

        @pl.when(c == 1)
        def _():
            pltpu.sync_copy(acc_sh.at[rows], pb_hbm.at[rows])

    return run(srcp, dstp, z16)


_TB = 1024
_TG = NP // _TB


def _scale_x_body(d0, d1, x, xs0, xs1, dv):
    deg = d0[:, :1] + d1[:, :1] + 1.0
    di = lax.rsqrt(deg)
    dv[...] = di
    xv = x[...] * di
    xs0[...] = xv[:, :128]
    xs1[...] = xv[:, 128:]


def _scale_x(d0a, d1a, xp):
    return pl.pallas_call(
        _scale_x_body,
        grid=(_TG,),
        in_specs=[
            pl.BlockSpec((_TB, 128), lambda i: (i, 0)),
            pl.BlockSpec((_TB, 128), lambda i: (i, 0)),
            pl.BlockSpec((_TB, D_IN), lambda i: (i, 0)),
        ],
        out_specs=[
            pl.BlockSpec((_TB, 128), lambda i: (i, 0)),
            pl.BlockSpec((_TB, 128), lambda i: (i, 0)),
            pl.BlockSpec((_TB, 1), lambda i: (i, 0)),
        ],
        out_shape=[
            jax.ShapeDtypeStruct((NP, 128), F32),
            jax.ShapeDtypeStruct((NP, 128), F32),
            jax.ShapeDtypeStruct((NP, 1), F32),
        ],
    )(d0a, d1a, xp)


def _layer1_body(p0, p1, dv, w1, b1, lnw, lnb, w2, g0, g1, g2, g3):
    di = dv[...]
    t = jnp.concatenate([p0[...], p1[...]], axis=1) * di
    u = jnp.dot(t, w1[...], preferred_element_type=F32, precision=HIGH)
    u = u + b1[...]
    mu = jnp.mean(u, axis=-1, keepdims=True)
    var = jnp.mean((u - mu) ** 2, axis=-1, keepdims=True)
    h = (u - mu) * lax.rsqrt(var + 1e-5) * lnw[...] + lnb[...]
    h = jnp.maximum(h, 0.0) * di
    g = jnp.dot(h, w2[...], preferred_element_type=F32, precision=HIGH)
    g0[...] = g[:, 0:128]
    g1[...] = g[:, 128:256]
    g2[...] = g[:, 256:384]
    g3[...] = g[:, 384:512]


def _layer1(p0, p1, dv, w1, b1, lnw, lnb, w2):
    full = lambda r, c: pl.BlockSpec((r, c), lambda i: (0, 0))
    return pl.pallas_call(
        _layer1_body,
        grid=(_TG,),
        in_specs=[
            pl.BlockSpec((_TB, 128), lambda i: (i, 0)),
            pl.BlockSpec((_TB, 128), lambda i: (i, 0)),
            pl.BlockSpec((_TB, 1), lambda i: (i, 0)),
            full(D_IN, D_H), full(1, D_H), full(1, D_H), full(1, D_H),
            full(D_H, D_H),
        ],
        out_specs=[pl.BlockSpec((_TB, 128), lambda i: (i, 0))] * 4,
        out_shape=[jax.ShapeDtypeStruct((NP, 128), F32)] * 4,
    )(p0, p1, dv, w1, b1, lnw, lnb, w2)


def _mid_body(q0, q1, q2, q3, dv, b, w, o0, o1, o2, o3):
    di = dv[...]
    q = jnp.concatenate([q0[...], q1[...], q2[...], q3[...]], axis=1)
    h = jnp.maximum(q * di + b[...], 0.0) * di
    g = jnp.dot(h, w[...], preferred_element_type=F32, precision=HIGH)
    o0[...] = g[:, 0:128]
    o1[...] = g[:, 128:256]
    o2[...] = g[:, 256:384]
    o3[...] = g[:, 384:512]


def _mid(q, dv, b, w):
    full = lambda r, c: pl.BlockSpec((r, c), lambda i: (0, 0))
    return pl.pallas_call(
        _mid_body,
        grid=(_TG,),
        in_specs=[pl.BlockSpec((_TB, 128), lambda i: (i, 0))] * 4 + [
            pl.BlockSpec((_TB, 1), lambda i: (i, 0)),
            full(1, D_H), full(D_H, D_H),
        ],
        out_specs=[pl.BlockSpec((_TB, 128), lambda i: (i, 0))] * 4,
        out_shape=[jax.ShapeDtypeStruct((NP, 128), F32)] * 4,
    )(*q, dv, b, w)


def _last_body(r0, r1, r2, r3, dv, b, w4, z16):
    di = dv[...]
    r = jnp.concatenate([r0[...], r1[...], r2[...], r3[...]], axis=1)
    h = jnp.maximum(r * di + b[...], 0.0) * di
    z = jnp.dot(h, w4[...], preferred_element_type=F32, precision=HIGH)
    z16[...] = jnp.broadcast_to(z, (z.shape[0], 128))


def _last(r, dv, b, w4):
    full = lambda rr, cc: pl.BlockSpec((rr, cc), lambda i: (0, 0))
    return pl.pallas_call(
        _last_body,
        grid=(_TG,),
        in_specs=[pl.BlockSpec((_TB, 128), lambda i: (i, 0))] * 4 + [
            pl.BlockSpec((_TB, 1), lambda i: (i, 0)),
            full(1, D_H), full(D_H, 1),
        ],
        out_specs=pl.BlockSpec((_TB, 128), lambda i: (i, 0)),
        out_shape=jax.ShapeDtypeStruct((NP, 128), F32),
    )(*r, dv, b, w4)


def _fin_body(pa, pb, z16, dv, b4, y):
    y[...] = (pa[:, :1] + pb[:, :1] - z16[:, :1]) * dv[...] + b4[...]


def _fin(pa, pb, z16, dv, b4):
    fb = 1000
    return pl.pallas_call(
        _fin_body,
        grid=(N // fb,),
        in_specs=[
            pl.BlockSpec((fb, 128), lambda i: (i, 0)),
            pl.BlockSpec((fb, 128), lambda i: (i, 0)),
            pl.BlockSpec((fb, 128), lambda i: (i, 0)),
            pl.BlockSpec((fb, 1), lambda i: (i, 0)),
            pl.BlockSpec((1, 1), lambda i: (0, 0)),
        ],
        out_specs=pl.BlockSpec((fb, 1), lambda i: (i, 0)),
        out_shape=jax.ShapeDtypeStruct((N, 1), F32),
    )(pa, pb, z16, dv, b4)


def kernel(x, edge_index, W1, b1, ln_w, ln_b, W2, b2, W3, b3, W4, b4):
    src = edge_index[0].astype(jnp.int32)
    dst = edge_index[1].astype(jnp.int32)
    srcp = jnp.concatenate([src, jnp.zeros((EP - E,), jnp.int32)])
    dstp = jnp.concatenate([dst, jnp.full((EP - E,), NP - 1, jnp.int32)])
    xp = jnp.zeros((NP, D_IN), F32).at[:N].set(x.astype(F32))

    dst2d = dstp.reshape(EP // IB, IB)
    d0, d1 = _sc_degree(dstp)
    xs0, xs1, dv = _scale_x(d0, d1, xp)
    p0, p1 = _sc_prop(srcp, dst2d, (xs0, xs1))
    g = _layer1(p0, p1, dv, W1, b1.reshape(1, -1),
                ln_w.reshape(1, -1), ln_b.reshape(1, -1), W2)
    q = _sc_prop(srcp, dst2d, g)
    g2 = _mid(q, dv, b2.reshape(1, -1), W3)
    r = _sc_prop(srcp, dst2d, g2)
    z16 = _last(r, dv, b3.reshape(1, -1), W4)
    pa, pb = _sc_prop_w(srcp, dstp, z16)
    y = _fin(pa, pb, z16, dv, b4.reshape(1, 1))
    return jnp.squeeze(y, -1)


# speedup vs baseline: 6.2600x; 1.1432x over previous
import functools

import jax
import jax.numpy as jnp
from jax import lax
from jax.experimental import pallas as pl
from jax.experimental.pallas import tpu as pltpu
from jax.experimental.pallas import tpu_sc as plsc

N = 10000
NP = 10240
E = 160000
EP = 163840
D_IN = 256
D_H = 512
L = 16
NC = 2
NS = 16
RPT = NP // NS
IB = 128
HIGH = lax.Precision.HIGHEST
F32 = jnp.float32


def _mesh():
    return plsc.VectorSubcoreMesh(core_axis_name="c", subcore_axis_name="s",
                                  num_cores=NC, num_subcores=NS)


def _fill_rows(ref, nrows, val, ncols=L):
    def body(i, carry):
        for j in range(ncols // L):
            ref[i, pl.ds(j * L, L)] = jnp.full((L,), val, F32)
        return carry
    lax.fori_loop(0, nrows, body, 0)


def _sc_degree(dstp):
    ept = EP // (NC * NS)
    nb = ept // IB
    fr = 64

    @functools.partial(
        pl.kernel,
        out_type=(jax.ShapeDtypeStruct((NP, 128), F32),
                  jax.ShapeDtypeStruct((NP, 128), F32)),
        mesh=_mesh(),
        scratch_types=[
            pltpu.VMEM((IB,), jnp.int32),
            pltpu.VMEM((IB, 128), F32),
            pltpu.VMEM((fr, 128), F32),
            pltpu.VMEM_SHARED((NP, 128), F32),
        ],
    )
    def run(dst_hbm, d0_hbm, d1_hbm, idx_v, ones_v, fill_v, acc_sh):
        c = lax.axis_index("c")
        s = lax.axis_index("s")
        _fill_rows(ones_v, IB, 1.0, ncols=128)
        _fill_rows(fill_v, fr, 0.0, ncols=128)
        for r in range(RPT // fr):
            pltpu.sync_copy(fill_v,
                            acc_sh.at[pl.ds(s * RPT + r * fr, fr)])
        plsc.subcore_barrier()
        base = c * (EP // NC) + s * ept

        def batch(k, carry):
            pltpu.sync_copy(dst_hbm.at[pl.ds(base + k * IB, IB)], idx_v)
            pltpu.sync_copy(ones_v, acc_sh.at[idx_v], add=True)
            return carry

        lax.fori_loop(0, nb, batch, 0)
        plsc.subcore_barrier()
        rows = pl.ds(s * RPT, RPT)

        @pl.when(c == 0)
        def _():
            pltpu.sync_copy(acc_sh.at[rows], d0_hbm.at[rows])

        @pl.when(c == 1)
        def _():
            pltpu.sync_copy(acc_sh.at[rows], d1_hbm.at[rows])

    return run(dstp)


def _edge_loop(src_hbm, tbl, acc_sh, didx_v, sidxA, sidxB,
               rows0, rows1, sem0, sem1, base, nb):
    nbq = nb // 4
    pltpu.sync_copy(src_hbm.at[pl.ds(base, 2 * IB)], sidxA)
    pltpu.async_copy(tbl.at[sidxA.at[pl.ds(0, IB)]], rows0, sem0)
    pltpu.async_copy(tbl.at[sidxA.at[pl.ds(IB, IB)]], rows1, sem1)

    def body4(j, carry):
        b0 = 4 * j
        pltpu.sync_copy(src_hbm.at[pl.ds(base + (b0 + 2) * IB, 2 * IB)],
                        sidxB)
        pltpu.make_async_copy(tbl.at[sidxA.at[pl.ds(0, IB)]],
                              rows0, sem0).wait()
        pltpu.sync_copy(rows0, acc_sh.at[didx_v.at[b0]], add=True)
        pltpu.async_copy(tbl.at[sidxB.at[pl.ds(0, IB)]], rows0, sem0)
        pltpu.make_async_copy(tbl.at[sidxA.at[pl.ds(IB, IB)]],
                              rows1, sem1).wait()
        pltpu.sync_copy(rows1, acc_sh.at[didx_v.at[b0 + 1]], add=True)
        pltpu.async_copy(tbl.at[sidxB.at[pl.ds(IB, IB)]], rows1, sem1)

        @pl.when(j < nbq - 1)
        def _():
            pltpu.sync_copy(
                src_hbm.at[pl.ds(base + (b0 + 4) * IB, 2 * IB)], sidxA)

        pltpu.make_async_copy(tbl.at[sidxB.at[pl.ds(0, IB)]],
                              rows0, sem0).wait()
        pltpu.sync_copy(rows0, acc_sh.at[didx_v.at[b0 + 2]], add=True)

        @pl.when(j < nbq - 1)
        def _():
            pltpu.async_copy(tbl.at[sidxA.at[pl.ds(0, IB)]], rows0, sem0)

        pltpu.make_async_copy(tbl.at[sidxB.at[pl.ds(IB, IB)]],
                              rows1, sem1).wait()
        pltpu.sync_copy(rows1, acc_sh.at[didx_v.at[b0 + 3]], add=True)

        @pl.when(j < nbq - 1)
        def _():
            pltpu.async_copy(tbl.at[sidxA.at[pl.ds(IB, IB)]], rows1, sem1)

        return carry

    lax.fori_loop(0, nbq, body4, 0)


def _sc_prop(srcp, dstp2d, tables):
    n = len(tables)
    cpc = n // NC
    ept = EP // NS
    nb = ept // IB
    nbh = nb // 2

    @functools.partial(
        pl.kernel,
        out_type=tuple(jax.ShapeDtypeStruct((NP, 128), F32)
                       for _ in range(n)),
        mesh=_mesh(),
        scratch_types=[
            pltpu.VMEM((2 * IB,), jnp.int32),
            pltpu.VMEM((2 * IB,), jnp.int32),
            pltpu.VMEM((nb, IB), jnp.int32),
            pltpu.VMEM((IB, 128), F32),
            pltpu.VMEM((IB, 128), F32),
            pltpu.VMEM_SHARED((NP, 128), F32),
            pltpu.SemaphoreType.DMA,
            pltpu.SemaphoreType.DMA,
        ],
    )
    def run(src_hbm, dst_hbm, *rest):
        tbls = rest[:n]
        outs = rest[n:2 * n]
        (sidxA, sidxB, didx_v, rows0, rows1,
         acc_sh, sem0, sem1) = rest[2 * n:]
        c = lax.axis_index("c")
        s = lax.axis_index("s")
        rows = pl.ds(s * RPT, RPT)
        base = s * ept
        pltpu.sync_copy(dst_hbm.at[pl.ds(s * nb, nb)], didx_v)

        for cc in range(NC):
            @pl.when(c == cc)
            def _(cc=cc):
                for k in range(cpc):
                    ch = cc * cpc + k
                    tbl = tbls[ch]
                    out = outs[ch]
                    pltpu.sync_copy(tbl.at[rows], acc_sh.at[rows])
                    plsc.subcore_barrier()
                    _edge_loop(src_hbm, tbl, acc_sh, didx_v, sidxA,
                               sidxB, rows0, rows1, sem0, sem1, base, nb)
                    plsc.subcore_barrier()
                    pltpu.sync_copy(acc_sh.at[rows], out.at[rows])
                    plsc.subcore_barrier()

    return run(srcp, dstp2d, *tables)


def _sc_prop_w(srcp, dstp2d, z16):
    ept = EP // (NC * NS)
    nb = ept // IB

    @functools.partial(
        pl.kernel,
        out_type=(jax.ShapeDtypeStruct((NP, 128), F32),
                  jax.ShapeDtypeStruct((NP, 128), F32)),
        mesh=_mesh(),
        scratch_types=[
            pltpu.VMEM((2 * IB,), jnp.int32),
            pltpu.VMEM((2 * IB,), jnp.int32),
            pltpu.VMEM((nb, IB), jnp.int32),
            pltpu.VMEM((IB, 128), F32),
            pltpu.VMEM((IB, 128), F32),
            pltpu.VMEM_SHARED((NP, 128), F32),
            pltpu.SemaphoreType.DMA,
            pltpu.SemaphoreType.DMA,
        ],
    )
    def run(src_hbm, dst_hbm, z_hbm, pa_hbm, pb_hbm,
            sidxA, sidxB, didx_v, rows0, rows1, acc_sh, sem0, sem1):
        c = lax.axis_index("c")
        s = lax.axis_index("s")
        rows = pl.ds(s * RPT, RPT)
        pltpu.sync_copy(z_hbm.at[rows], acc_sh.at[rows])
        base = c * (EP // NC) + s * ept
        drow = c * (EP // NC // IB) + s * nb
        pltpu.sync_copy(dst_hbm.at[pl.ds(drow, nb)], didx_v)
        plsc.subcore_barrier()
        _edge_loop(src_hbm, z_hbm, acc_sh, didx_v, sidxA, sidxB,
                   rows0, rows1, sem0, sem1, base, nb)
        plsc.subcore_barrier()

        @pl.when(c == 0)
        def _():
            pltpu.sync_copy(acc_sh.at[rows], pa_hbm.at[rows])

        @pl.when(c == 1)
        def _():
            pltpu.sync_copy(acc_sh.at[rows], pb_hbm.at[rows])

    return run(srcp, dstp2d, z16)


_TB = 1024
_TG = NP // _TB


def _scale_x_body(d0, d1, x, xs0, xs1, dv):
    deg = d0[:, :1] + d1[:, :1] + 1.0
    di = lax.rsqrt(deg)
    dv[...] = di
    xv = x[...] * di
    xs0[...] = xv[:, :128]
    xs1[...] = xv[:, 128:]


def _scale_x(d0a, d1a, xp):
    return pl.pallas_call(
        _scale_x_body,
        grid=(_TG,),
        in_specs=[
            pl.BlockSpec((_TB, 128), lambda i: (i, 0)),
            pl.BlockSpec((_TB, 128), lambda i: (i, 0)),
            pl.BlockSpec((_TB, D_IN), lambda i: (i, 0)),
        ],
        out_specs=[
            pl.BlockSpec((_TB, 128), lambda i: (i, 0)),
            pl.BlockSpec((_TB, 128), lambda i: (i, 0)),
            pl.BlockSpec((_TB, 1), lambda i: (i, 0)),
        ],
        out_shape=[
            jax.ShapeDtypeStruct((NP, 128), F32),
            jax.ShapeDtypeStruct((NP, 128), F32),
            jax.ShapeDtypeStruct((NP, 1), F32),
        ],
    )(d0a, d1a, xp)


def _layer1_body(p0, p1, dv, w1, b1, lnw, lnb, w2, g0, g1, g2, g3):
    di = dv[...]
    t = jnp.concatenate([p0[...], p1[...]], axis=1) * di
    u = jnp.dot(t, w1[...], preferred_element_type=F32, precision=HIGH)
    u = u + b1[...]
    mu = jnp.mean(u, axis=-1, keepdims=True)
    var = jnp.mean((u - mu) ** 2, axis=-1, keepdims=True)
    h = (u - mu) * lax.rsqrt(var + 1e-5) * lnw[...] + lnb[...]
    h = jnp.maximum(h, 0.0) * di
    g = jnp.dot(h, w2[...], preferred_element_type=F32, precision=HIGH)
    g0[...] = g[:, 0:128]
    g1[...] = g[:, 128:256]
    g2[...] = g[:, 256:384]
    g3[...] = g[:, 384:512]


def _layer1(p0, p1, dv, w1, b1, lnw, lnb, w2):
    full = lambda r, c: pl.BlockSpec((r, c), lambda i: (0, 0))
    return pl.pallas_call(
        _layer1_body,
        grid=(_TG,),
        in_specs=[
            pl.BlockSpec((_TB, 128), lambda i: (i, 0)),
            pl.BlockSpec((_TB, 128), lambda i: (i, 0)),
            pl.BlockSpec((_TB, 1), lambda i: (i, 0)),
            full(D_IN, D_H), full(1, D_H), full(1, D_H), full(1, D_H),
            full(D_H, D_H),
        ],
        out_specs=[pl.BlockSpec((_TB, 128), lambda i: (i, 0))] * 4,
        out_shape=[jax.ShapeDtypeStruct((NP, 128), F32)] * 4,
    )(p0, p1, dv, w1, b1, lnw, lnb, w2)


def _mid_body(q0, q1, q2, q3, dv, b, w, o0, o1, o2, o3):
    di = dv[...]
    q = jnp.concatenate([q0[...], q1[...], q2[...], q3[...]], axis=1)
    h = jnp.maximum(q * di + b[...], 0.0) * di
    g = jnp.dot(h, w[...], preferred_element_type=F32, precision=HIGH)
    o0[...] = g[:, 0:128]
    o1[...] = g[:, 128:256]
    o2[...] = g[:, 256:384]
    o3[...] = g[:, 384:512]


def _mid(q, dv, b, w):
    full = lambda r, c: pl.BlockSpec((r, c), lambda i: (0, 0))
    return pl.pallas_call(
        _mid_body,
        grid=(_TG,),
        in_specs=[pl.BlockSpec((_TB, 128), lambda i: (i, 0))] * 4 + [
            pl.BlockSpec((_TB, 1), lambda i: (i, 0)),
            full(1, D_H), full(D_H, D_H),
        ],
        out_specs=[pl.BlockSpec((_TB, 128), lambda i: (i, 0))] * 4,
        out_shape=[jax.ShapeDtypeStruct((NP, 128), F32)] * 4,
    )(*q, dv, b, w)


def _last_body(r0, r1, r2, r3, dv, b, w4, z16):
    di = dv[...]
    r = jnp.concatenate([r0[...], r1[...], r2[...], r3[...]], axis=1)
    h = jnp.maximum(r * di + b[...], 0.0) * di
    z = jnp.dot(h, w4[...], preferred_element_type=F32, precision=HIGH)
    z16[...] = jnp.broadcast_to(z, (z.shape[0], 128))


def _last(r, dv, b, w4):
    full = lambda rr, cc: pl.BlockSpec((rr, cc), lambda i: (0, 0))
    return pl.pallas_call(
        _last_body,
        grid=(_TG,),
        in_specs=[pl.BlockSpec((_TB, 128), lambda i: (i, 0))] * 4 + [
            pl.BlockSpec((_TB, 1), lambda i: (i, 0)),
            full(1, D_H), full(D_H, 1),
        ],
        out_specs=pl.BlockSpec((_TB, 128), lambda i: (i, 0)),
        out_shape=jax.ShapeDtypeStruct((NP, 128), F32),
    )(*r, dv, b, w4)


def _fin_body(pa, pb, z16, dv, b4, y):
    y[...] = (pa[:, :1] + pb[:, :1] - z16[:, :1]) * dv[...] + b4[...]


def _fin(pa, pb, z16, dv, b4):
    fb = 1000
    return pl.pallas_call(
        _fin_body,
        grid=(N // fb,),
        in_specs=[
            pl.BlockSpec((fb, 128), lambda i: (i, 0)),
            pl.BlockSpec((fb, 128), lambda i: (i, 0)),
            pl.BlockSpec((fb, 128), lambda i: (i, 0)),
            pl.BlockSpec((fb, 1), lambda i: (i, 0)),
            pl.BlockSpec((1, 1), lambda i: (0, 0)),
        ],
        out_specs=pl.BlockSpec((fb, 1), lambda i: (i, 0)),
        out_shape=jax.ShapeDtypeStruct((N, 1), F32),
    )(pa, pb, z16, dv, b4)


def kernel(x, edge_index, W1, b1, ln_w, ln_b, W2, b2, W3, b3, W4, b4):
    src = edge_index[0].astype(jnp.int32)
    dst = edge_index[1].astype(jnp.int32)
    srcp = jnp.concatenate([src, jnp.zeros((EP - E,), jnp.int32)])
    dstp = jnp.concatenate([dst, jnp.full((EP - E,), NP - 1, jnp.int32)])
    xp = jnp.zeros((NP, D_IN), F32).at[:N].set(x.astype(F32))

    dst2d = dstp.reshape(EP // IB, IB)
    d0, d1 = _sc_degree(dstp)
    xs0, xs1, dv = _scale_x(d0, d1, xp)
    p0, p1 = _sc_prop(srcp, dst2d, (xs0, xs1))
    g = _layer1(p0, p1, dv, W1, b1.reshape(1, -1),
                ln_w.reshape(1, -1), ln_b.reshape(1, -1), W2)
    q = _sc_prop(srcp, dst2d, g)
    g2 = _mid(q, dv, b2.reshape(1, -1), W3)
    r = _sc_prop(srcp, dst2d, g2)
    z16 = _last(r, dv, b3.reshape(1, -1), W4)
    pa, pb = _sc_prop_w(srcp, dst2d, z16)
    y = _fin(pa, pb, z16, dv, b4.reshape(1, 1))
    return jnp.squeeze(y, -1)


# speedup vs baseline: 6.2610x; 1.0002x over previous
import functools

import jax
import jax.numpy as jnp
from jax import lax
from jax.experimental import pallas as pl
from jax.experimental.pallas import tpu as pltpu
from jax.experimental.pallas import tpu_sc as plsc

N = 10000
NP = 10240
E = 160000
EP = 163840
D_IN = 256
D_H = 512
L = 16
NC = 2
NS = 16
RPT = NP // NS
IB = 128
HIGH = lax.Precision.HIGHEST
F32 = jnp.float32


def _mesh():
    return plsc.VectorSubcoreMesh(core_axis_name="c", subcore_axis_name="s",
                                  num_cores=NC, num_subcores=NS)


def _fill_rows(ref, nrows, val, ncols=L):
    def body(i, carry):
        for j in range(ncols // L):
            ref[i, pl.ds(j * L, L)] = jnp.full((L,), val, F32)
        return carry
    lax.fori_loop(0, nrows, body, 0)


def _sc_degree(dstp):
    ept = EP // (NC * NS)
    nb = ept // IB
    fr = 64

    @functools.partial(
        pl.kernel,
        out_type=(jax.ShapeDtypeStruct((NP, 128), F32),
                  jax.ShapeDtypeStruct((NP, 128), F32)),
        mesh=_mesh(),
        scratch_types=[
            pltpu.VMEM((IB,), jnp.int32),
            pltpu.VMEM((IB, 128), F32),
            pltpu.VMEM((fr, 128), F32),
            pltpu.VMEM_SHARED((NP, 128), F32),
        ],
    )
    def run(dst_hbm, d0_hbm, d1_hbm, idx_v, ones_v, fill_v, acc_sh):
        c = lax.axis_index("c")
        s = lax.axis_index("s")
        _fill_rows(ones_v, IB, 1.0, ncols=128)
        _fill_rows(fill_v, fr, 0.0, ncols=128)
        for r in range(RPT // fr):
            pltpu.sync_copy(fill_v,
                            acc_sh.at[pl.ds(s * RPT + r * fr, fr)])
        plsc.subcore_barrier()
        base = c * (EP // NC) + s * ept

        def batch(k, carry):
            pltpu.sync_copy(dst_hbm.at[pl.ds(base + k * IB, IB)], idx_v)
            pltpu.sync_copy(ones_v, acc_sh.at[idx_v], add=True)
            return carry

        lax.fori_loop(0, nb, batch, 0)
        plsc.subcore_barrier()
        rows = pl.ds(s * RPT, RPT)

        @pl.when(c == 0)
        def _():
            pltpu.sync_copy(acc_sh.at[rows], d0_hbm.at[rows])

        @pl.when(c == 1)
        def _():
            pltpu.sync_copy(acc_sh.at[rows], d1_hbm.at[rows])

    return run(dstp)


def _edge_loop(src_hbm, tbl, acc_sh, didx_v, sidxA, sidxB,
               rows0, rows1, sem0, sem1, base, nb):
    nbq = nb // 4
    pltpu.sync_copy(src_hbm.at[pl.ds(base, 2 * IB)], sidxA)
    pltpu.async_copy(tbl.at[sidxA.at[pl.ds(0, IB)]], rows0, sem0)
    pltpu.async_copy(tbl.at[sidxA.at[pl.ds(IB, IB)]], rows1, sem1)

    def body4(j, carry):
        b0 = 4 * j
        pltpu.sync_copy(src_hbm.at[pl.ds(base + (b0 + 2) * IB, 2 * IB)],
                        sidxB)
        pltpu.make_async_copy(tbl.at[sidxA.at[pl.ds(0, IB)]],
                              rows0, sem0).wait()
        pltpu.sync_copy(rows0, acc_sh.at[didx_v.at[b0]], add=True)
        pltpu.async_copy(tbl.at[sidxB.at[pl.ds(0, IB)]], rows0, sem0)
        pltpu.make_async_copy(tbl.at[sidxA.at[pl.ds(IB, IB)]],
                              rows1, sem1).wait()
        pltpu.sync_copy(rows1, acc_sh.at[didx_v.at[b0 + 1]], add=True)
        pltpu.async_copy(tbl.at[sidxB.at[pl.ds(IB, IB)]], rows1, sem1)

        @pl.when(j < nbq - 1)
        def _():
            pltpu.sync_copy(
                src_hbm.at[pl.ds(base + (b0 + 4) * IB, 2 * IB)], sidxA)

        pltpu.make_async_copy(tbl.at[sidxB.at[pl.ds(0, IB)]],
                              rows0, sem0).wait()
        pltpu.sync_copy(rows0, acc_sh.at[didx_v.at[b0 + 2]], add=True)

        @pl.when(j < nbq - 1)
        def _():
            pltpu.async_copy(tbl.at[sidxA.at[pl.ds(0, IB)]], rows0, sem0)

        pltpu.make_async_copy(tbl.at[sidxB.at[pl.ds(IB, IB)]],
                              rows1, sem1).wait()
        pltpu.sync_copy(rows1, acc_sh.at[didx_v.at[b0 + 3]], add=True)

        @pl.when(j < nbq - 1)
        def _():
            pltpu.async_copy(tbl.at[sidxA.at[pl.ds(IB, IB)]], rows1, sem1)

        return carry

    lax.fori_loop(0, nbq, body4, 0)


def _sc_prop(srcp, dstp2d, tables):
    n = len(tables)
    cpc = n // NC
    ept = EP // NS
    nb = ept // IB

    @functools.partial(
        pl.kernel,
        out_type=tuple(jax.ShapeDtypeStruct((NP, 128), F32)
                       for _ in range(n)),
        mesh=_mesh(),
        scratch_types=[
            pltpu.VMEM((2 * IB,), jnp.int32),
            pltpu.VMEM((2 * IB,), jnp.int32),
            pltpu.VMEM((nb, IB), jnp.int32),
            pltpu.VMEM((IB, 128), F32),
            pltpu.VMEM((IB, 128), F32),
            pltpu.VMEM_SHARED((NP, 128), F32),
            pltpu.SemaphoreType.DMA,
            pltpu.SemaphoreType.DMA,
        ],
    )
    def run(src_hbm, dst_hbm, *rest):
        tbls = rest[:n]
        outs = rest[n:2 * n]
        (sidxA, sidxB, didx_v, rows0, rows1,
         acc_sh, sem0, sem1) = rest[2 * n:]
        c = lax.axis_index("c")
        s = lax.axis_index("s")
        rows = pl.ds(s * RPT, RPT)
        base = s * ept
        pltpu.sync_copy(dst_hbm.at[pl.ds(s * nb, nb)], didx_v)

        for cc in range(NC):
            @pl.when(c == cc)
            def _(cc=cc):
                for k in range(cpc):
                    ch = cc * cpc + k
                    tbl = tbls[ch]
                    out = outs[ch]
                    pltpu.sync_copy(tbl.at[rows], acc_sh.at[rows])
                    plsc.subcore_barrier()
                    _edge_loop(src_hbm, tbl, acc_sh, didx_v, sidxA,
                               sidxB, rows0, rows1, sem0, sem1, base, nb)
                    plsc.subcore_barrier()
                    pltpu.sync_copy(acc_sh.at[rows], out.at[rows])
                    plsc.subcore_barrier()

    return run(srcp, dstp2d, *tables)


def _sc_prop_w(srcp, dstp2d, z16):
    ept = EP // (NC * NS)
    nb = ept // IB

    @functools.partial(
        pl.kernel,
        out_type=(jax.ShapeDtypeStruct((NP, 128), F32),
                  jax.ShapeDtypeStruct((NP, 128), F32)),
        mesh=_mesh(),
        scratch_types=[
            pltpu.VMEM((2 * IB,), jnp.int32),
            pltpu.VMEM((2 * IB,), jnp.int32),
            pltpu.VMEM((nb, IB), jnp.int32),
            pltpu.VMEM((IB, 128), F32),
            pltpu.VMEM((IB, 128), F32),
            pltpu.VMEM_SHARED((NP, 128), F32),
            pltpu.SemaphoreType.DMA,
            pltpu.SemaphoreType.DMA,
        ],
    )
    def run(src_hbm, dst_hbm, z_hbm, pa_hbm, pb_hbm,
            sidxA, sidxB, didx_v, rows0, rows1, acc_sh, sem0, sem1):
        c = lax.axis_index("c")
        s = lax.axis_index("s")
        rows = pl.ds(s * RPT, RPT)
        pltpu.sync_copy(z_hbm.at[rows], acc_sh.at[rows])
        base = c * (EP // NC) + s * ept
        drow = c * (EP // NC // IB) + s * nb
        pltpu.sync_copy(dst_hbm.at[pl.ds(drow, nb)], didx_v)
        plsc.subcore_barrier()
        _edge_loop(src_hbm, z_hbm, acc_sh, didx_v, sidxA, sidxB,
                   rows0, rows1, sem0, sem1, base, nb)
        plsc.subcore_barrier()

        @pl.when(c == 0)
        def _():
            pltpu.sync_copy(acc_sh.at[rows], pa_hbm.at[rows])

        @pl.when(c == 1)
        def _():
            pltpu.sync_copy(acc_sh.at[rows], pb_hbm.at[rows])

    return run(srcp, dstp2d, z16)


_TB = 1024
_TG = NP // _TB


def _scale_x_body(d0, d1, x, xs0, xs1, dv):
    deg = d0[:, :1] + d1[:, :1] + 1.0
    di = lax.rsqrt(deg)
    dv[...] = di
    xv = x[...] * di
    xs0[...] = xv[:, :128]
    xs1[...] = xv[:, 128:]


def _scale_x(d0a, d1a, xp):
    return pl.pallas_call(
        _scale_x_body,
        grid=(_TG,),
        in_specs=[
            pl.BlockSpec((_TB, 128), lambda i: (i, 0)),
            pl.BlockSpec((_TB, 128), lambda i: (i, 0)),
            pl.BlockSpec((_TB, D_IN), lambda i: (i, 0)),
        ],
        out_specs=[
            pl.BlockSpec((_TB, 128), lambda i: (i, 0)),
            pl.BlockSpec((_TB, 128), lambda i: (i, 0)),
            pl.BlockSpec((_TB, 1), lambda i: (i, 0)),
        ],
        out_shape=[
            jax.ShapeDtypeStruct((NP, 128), F32),
            jax.ShapeDtypeStruct((NP, 128), F32),
            jax.ShapeDtypeStruct((NP, 1), F32),
        ],
    )(d0a, d1a, xp)


def _layer1_body(p0, p1, dv, w1, b1, lnw, lnb, w2, g0, g1, g2, g3):
    di = dv[...]
    t = jnp.concatenate([p0[...], p1[...]], axis=1) * di
    u = jnp.dot(t, w1[...], preferred_element_type=F32, precision=HIGH)
    u = u + b1[...]
    mu = jnp.mean(u, axis=-1, keepdims=True)
    var = jnp.mean((u - mu) ** 2, axis=-1, keepdims=True)
    h = (u - mu) * lax.rsqrt(var + 1e-5) * lnw[...] + lnb[...]
    h = jnp.maximum(h, 0.0) * di
    g = jnp.dot(h, w2[...], preferred_element_type=F32, precision=HIGH)
    g0[...] = g[:, 0:128]
    g1[...] = g[:, 128:256]
    g2[...] = g[:, 256:384]
    g3[...] = g[:, 384:512]


def _layer1(p0, p1, dv, w1, b1, lnw, lnb, w2):
    full = lambda r, c: pl.BlockSpec((r, c), lambda i: (0, 0))
    return pl.pallas_call(
        _layer1_body,
        grid=(_TG,),
        in_specs=[
            pl.BlockSpec((_TB, 128), lambda i: (i, 0)),
            pl.BlockSpec((_TB, 128), lambda i: (i, 0)),
            pl.BlockSpec((_TB, 1), lambda i: (i, 0)),
            full(D_IN, D_H), full(1, D_H), full(1, D_H), full(1, D_H),
            full(D_H, D_H),
        ],
        out_specs=[pl.BlockSpec((_TB, 128), lambda i: (i, 0))] * 4,
        out_shape=[jax.ShapeDtypeStruct((NP, 128), F32)] * 4,
    )(p0, p1, dv, w1, b1, lnw, lnb, w2)


def _mid_body(q0, q1, q2, q3, dv, b, w, o0, o1, o2, o3):
    di = dv[...]
    q = jnp.concatenate([q0[...], q1[...], q2[...], q3[...]], axis=1)
    h = jnp.maximum(q * di + b[...], 0.0) * di
    g = jnp.dot(h, w[...], preferred_element_type=F32, precision=HIGH)
    o0[...] = g[:, 0:128]
    o1[...] = g[:, 128:256]
    o2[...] = g[:, 256:384]
    o3[...] = g[:, 384:512]


def _mid(q, dv, b, w):
    full = lambda r, c: pl.BlockSpec((r, c), lambda i: (0, 0))
    return pl.pallas_call(
        _mid_body,
        grid=(_TG,),
        in_specs=[pl.BlockSpec((_TB, 128), lambda i: (i, 0))] * 4 + [
            pl.BlockSpec((_TB, 1), lambda i: (i, 0)),
            full(1, D_H), full(D_H, D_H),
        ],
        out_specs=[pl.BlockSpec((_TB, 128), lambda i: (i, 0))] * 4,
        out_shape=[jax.ShapeDtypeStruct((NP, 128), F32)] * 4,
    )(*q, dv, b, w)


def _last_body(r0, r1, r2, r3, dv, b, w4, z16):
    di = dv[...]
    r = jnp.concatenate([r0[...], r1[...], r2[...], r3[...]], axis=1)
    h = jnp.maximum(r * di + b[...], 0.0) * di
    z = jnp.dot(h, w4[...], preferred_element_type=F32, precision=HIGH)
    z16[...] = jnp.broadcast_to(z, (z.shape[0], 128))


def _last(r, dv, b, w4):
    full = lambda rr, cc: pl.BlockSpec((rr, cc), lambda i: (0, 0))
    return pl.pallas_call(
        _last_body,
        grid=(_TG,),
        in_specs=[pl.BlockSpec((_TB, 128), lambda i: (i, 0))] * 4 + [
            pl.BlockSpec((_TB, 1), lambda i: (i, 0)),
            full(1, D_H), full(D_H, 1),
        ],
        out_specs=pl.BlockSpec((_TB, 128), lambda i: (i, 0)),
        out_shape=jax.ShapeDtypeStruct((NP, 128), F32),
    )(*r, dv, b, w4)


def _fin_body(pa, pb, z16, dv, b4, y):
    y[...] = (pa[:, :1] + pb[:, :1] - z16[:, :1]) * dv[...] + b4[...]


def _fin(pa, pb, z16, dv, b4):
    fb = 1000
    return pl.pallas_call(
        _fin_body,
        grid=(N // fb,),
        in_specs=[
            pl.BlockSpec((fb, 128), lambda i: (i, 0)),
            pl.BlockSpec((fb, 128), lambda i: (i, 0)),
            pl.BlockSpec((fb, 128), lambda i: (i, 0)),
            pl.BlockSpec((fb, 1), lambda i: (i, 0)),
            pl.BlockSpec((1, 1), lambda i: (0, 0)),
        ],
        out_specs=pl.BlockSpec((fb, 1), lambda i: (i, 0)),
        out_shape=jax.ShapeDtypeStruct((N, 1), F32),
    )(pa, pb, z16, dv, b4)


def kernel(x, edge_index, W1, b1, ln_w, ln_b, W2, b2, W3, b3, W4, b4):
    src = edge_index[0].astype(jnp.int32)
    dst = edge_index[1].astype(jnp.int32)
    srcp = jnp.concatenate([src, jnp.zeros((EP - E,), jnp.int32)])
    dstp = jnp.concatenate([dst, jnp.full((EP - E,), NP - 1, jnp.int32)])
    xp = jnp.zeros((NP, D_IN), F32).at[:N].set(x.astype(F32))

    dst2d = dstp.reshape(EP // IB, IB)
    d0, d1 = _sc_degree(dstp)
    xs0, xs1, dv = _scale_x(d0, d1, xp)
    p0, p1 = _sc_prop(srcp, dst2d, (xs0, xs1))
    g = _layer1(p0, p1, dv, W1, b1.reshape(1, -1),
                ln_w.reshape(1, -1), ln_b.reshape(1, -1), W2)
    q = _sc_prop(srcp, dst2d, g)
    g2 = _mid(q, dv, b2.reshape(1, -1), W3)
    r = _sc_prop(srcp, dst2d, g2)
    z16 = _last(r, dv, b3.reshape(1, -1), W4)
    pa, pb = _sc_prop_w(srcp, dst2d, z16)
    y = _fin(pa, pb, z16, dv, b4.reshape(1, 1))
    return jnp.squeeze(y, -1)
